# baseline (speedup 1.0000x reference)
import functools

import jax
import jax.numpy as jnp
from jax import lax
from jax.experimental import pallas as pl
from jax.experimental.pallas import tpu as pltpu
from jax.experimental.pallas import tpu_sc as plsc

VOCAB_SIZE = 100000
D_MODEL = 1024
MAX_LEN = 8192
BATCH = 4
SEQ_LEN = 4096

NUM_CORES = 2
NUM_SUBCORES = 16
NUM_WORKERS = NUM_CORES * NUM_SUBCORES
N_ROWS = BATCH * SEQ_LEN
S_BLOCK = SEQ_LEN // NUM_WORKERS
CHUNK = 16
N_PCHUNKS = S_BLOCK // CHUNK
LANES = 16
GROUPS = D_MODEL // LANES
LAST_I = N_PCHUNKS - 2


def _body(x_hbm, tok_hbm, pos_hbm, out_hbm, idx0, idx1, idx2, idx3,
          tok0, tok1, pos0, pos1, sg0, sg1, sp0, sp1, so0, so1):
    wid = lax.axis_index("s") * NUM_CORES + lax.axis_index("c")
    s_base = wid * S_BLOCK

    idxs = (idx0, idx1, idx2, idx3)
    toks = (tok0, tok1)
    poss = (pos0, pos1)
    sgs = (sg0, sg1)
    sos = (so0, so1)

    for b in range(BATCH):
        pltpu.sync_copy(x_hbm.at[b, pl.ds(s_base, S_BLOCK)], idxs[b])

    def start_gather(c, b, tb):
        pltpu.async_copy(
            tok_hbm.at[idxs[b].at[pl.ds(c * CHUNK, CHUNK)]],
            toks[tb], sgs[tb])

    def wait_gather(tb):
        pltpu.make_async_copy(tok_hbm.at[pl.ds(0, CHUNK)], toks[tb], sgs[tb]).wait()

    sps = (sp0, sp1)

    def start_pos(c, pb):
        pltpu.async_copy(pos_hbm.at[pl.ds(s_base + c * CHUNK, CHUNK)],
                         poss[pb], sps[pb])

    def wait_pos(pb):
        pltpu.make_async_copy(pos_hbm.at[pl.ds(0, CHUNK)], poss[pb], sps[pb]).wait()

    def start_out(c, b, tb):
        pltpu.async_copy(
            toks[tb],
            out_hbm.at[pl.ds(b * SEQ_LEN + s_base + c * CHUNK, CHUNK)],
            sos[tb])

    def wait_out(tb):
        pltpu.make_async_copy(toks[tb], out_hbm.at[pl.ds(0, CHUNK)], sos[tb]).wait()

    def add_chunk(tb, pb):
        tok_v, pos_v = toks[tb], poss[pb]

        def row_add(i, _):
            for grp in range(GROUPS):
                sl = pl.ds(grp * LANES, LANES)
                tok_v[i, sl] = tok_v[i, sl] + pos_v[i, sl]
            return 0

        lax.fori_loop(0, CHUNK, row_add, 0, unroll=False)

    start_pos(0, 0)
    start_gather(0, 0, 0)

    @pl.loop(0, N_PCHUNKS, step=2)
    def _chunk_pair(i):
        for cc in (0, 1):
            c = i + cc
            pb = cc
            if cc == 0:
                start_pos(c + 1, 1 - pb)
            else:
                @pl.when(i < LAST_I)
                def _():
                    start_pos(c + 1, 1 - pb)

            wait_pos(pb)

            for b in range(BATCH):
                tb = b % 2
                nt = 1 - tb
                if cc == 0 and b == 0:
                    @pl.when(i > 0)
                    def _():
                        wait_out(nt)
                elif cc == 1 and b == BATCH - 1:
                    @pl.when(i < LAST_I)
                    def _():
                        wait_out(nt)
                else:
                    wait_out(nt)

                if b < BATCH - 1:
                    start_gather(c, b + 1, nt)
                elif cc == 0:
                    start_gather(c + 1, 0, nt)
                else:
                    @pl.when(i < LAST_I)
                    def _():
                        start_gather(c + 1, 0, nt)

                wait_gather(tb)
                add_chunk(tb, pb)
                start_out(c, b, tb)

    wait_out(0)
    wait_out(1)


@jax.jit
def _run(x2d, token_table, position_table):
    mesh = plsc.VectorSubcoreMesh(core_axis_name="c", subcore_axis_name="s")
    k = pl.kernel(
        _body,
        out_type=jax.ShapeDtypeStruct((N_ROWS, D_MODEL), jnp.float32),
        mesh=mesh,
        scratch_types=[
            pltpu.VMEM((S_BLOCK,), jnp.int32),
            pltpu.VMEM((S_BLOCK,), jnp.int32),
            pltpu.VMEM((S_BLOCK,), jnp.int32),
            pltpu.VMEM((S_BLOCK,), jnp.int32),
            pltpu.VMEM((CHUNK, D_MODEL), jnp.float32),
            pltpu.VMEM((CHUNK, D_MODEL), jnp.float32),
            pltpu.VMEM((CHUNK, D_MODEL), jnp.float32),
            pltpu.VMEM((CHUNK, D_MODEL), jnp.float32),
            pltpu.SemaphoreType.DMA,
            pltpu.SemaphoreType.DMA,
            pltpu.SemaphoreType.DMA,
            pltpu.SemaphoreType.DMA,
            pltpu.SemaphoreType.DMA,
            pltpu.SemaphoreType.DMA,
        ],
    )
    return k(x2d, token_table, position_table)


def kernel(x, token_table, position_table):
    out = _run(x.astype(jnp.int32), token_table, position_table)
    return out.reshape(BATCH, SEQ_LEN, D_MODEL)


# speedup vs baseline: 1.2131x; 1.2131x over previous
import functools

import jax
import jax.numpy as jnp
from jax import lax
from jax.experimental import pallas as pl
from jax.experimental.pallas import tpu as pltpu
from jax.experimental.pallas import tpu_sc as plsc

VOCAB_SIZE = 100000
D_MODEL = 1024
MAX_LEN = 8192
BATCH = 4
SEQ_LEN = 4096

NUM_CORES = 2
NUM_SUBCORES = 16
NUM_WORKERS = NUM_CORES * NUM_SUBCORES
N_ROWS = BATCH * SEQ_LEN
S_BLOCK = SEQ_LEN // NUM_WORKERS
CHUNK = 16
N_PCHUNKS = S_BLOCK // CHUNK
N_JOBS = N_PCHUNKS * BATCH
LANES = 16
GROUPS = D_MODEL // LANES
LAST_I = N_PCHUNKS - 2


def _body(x_hbm, tok_hbm, pos_hbm, out_hbm, idx0, idx1, idx2, idx3,
          tok0, tok1, tok2, tok3, pos0, pos1,
          sg0, sg1, sg2, sg3, sp0, sp1, so0, so1, so2, so3):
    wid = lax.axis_index("s") * NUM_CORES + lax.axis_index("c")
    s_base = wid * S_BLOCK

    idxs = (idx0, idx1, idx2, idx3)
    toks = (tok0, tok1, tok2, tok3)
    poss = (pos0, pos1)
    sgs = (sg0, sg1, sg2, sg3)
    sps = (sp0, sp1)
    sos = (so0, so1, so2, so3)

    for b in range(BATCH):
        pltpu.sync_copy(x_hbm.at[b, pl.ds(s_base, S_BLOCK)], idxs[b])

    def start_gather(c, b):
        pltpu.async_copy(tok_hbm.at[idxs[b].at[pl.ds(c * CHUNK, CHUNK)]],
                         toks[b], sgs[b])

    def wait_gather(b):
        pltpu.make_async_copy(tok_hbm.at[pl.ds(0, CHUNK)], toks[b], sgs[b]).wait()

    def start_pos(c, pb):
        pltpu.async_copy(pos_hbm.at[pl.ds(s_base + c * CHUNK, CHUNK)],
                         poss[pb], sps[pb])

    def wait_pos(pb):
        pltpu.make_async_copy(pos_hbm.at[pl.ds(0, CHUNK)], poss[pb], sps[pb]).wait()

    def start_out(c, b):
        pltpu.async_copy(toks[b],
                         out_hbm.at[pl.ds(b * SEQ_LEN + s_base + c * CHUNK, CHUNK)],
                         sos[b])

    def wait_out(b):
        pltpu.make_async_copy(toks[b], out_hbm.at[pl.ds(0, CHUNK)], sos[b]).wait()

    def add_chunk(b, pb):
        tok_v, pos_v = toks[b], poss[pb]

        def row_add(i, _):
            for grp in range(GROUPS):
                sl = pl.ds(grp * LANES, LANES)
                tok_v[i, sl] = tok_v[i, sl] + pos_v[i, sl]
            return 0

        lax.fori_loop(0, CHUNK, row_add, 0, unroll=False)

    start_pos(0, 0)
    start_gather(0, 0)
    start_gather(0, 1)

    @pl.loop(0, N_PCHUNKS, step=2)
    def _chunk_pair(i):
        for cc in (0, 1):
            c = i + cc
            pb = cc
            if cc == 0:
                start_pos(c + 1, 1 - pb)
            else:
                @pl.when(i < LAST_I)
                def _():
                    start_pos(c + 1, 1 - pb)

            wait_pos(pb)

            for b in range(BATCH):
                b2 = (b + 2) % BATCH
                c2 = c + (1 if b >= 2 else 0)
                if cc == 0 and b < 2:
                    @pl.when(i > 0)
                    def _():
                        wait_out(b2)

                    start_gather(c2, b2)
                elif cc == 1 and b >= 2:
                    wait_out(b2)

                    @pl.when(i < LAST_I)
                    def _():
                        start_gather(c2, b2)
                else:
                    wait_out(b2)
                    start_gather(c2, b2)

                wait_gather(b)
                add_chunk(b, pb)
                start_out(c, b)

    wait_out(2)
    wait_out(3)


@jax.jit
def _run(x2d, token_table, position_table):
    mesh = plsc.VectorSubcoreMesh(core_axis_name="c", subcore_axis_name="s")
    k = pl.kernel(
        _body,
        out_type=jax.ShapeDtypeStruct((N_ROWS, D_MODEL), jnp.float32),
        mesh=mesh,
        scratch_types=[
            pltpu.VMEM((S_BLOCK,), jnp.int32),
            pltpu.VMEM((S_BLOCK,), jnp.int32),
            pltpu.VMEM((S_BLOCK,), jnp.int32),
            pltpu.VMEM((S_BLOCK,), jnp.int32),
            pltpu.VMEM((CHUNK, D_MODEL), jnp.float32),
            pltpu.VMEM((CHUNK, D_MODEL), jnp.float32),
            pltpu.VMEM((CHUNK, D_MODEL), jnp.float32),
            pltpu.VMEM((CHUNK, D_MODEL), jnp.float32),
            pltpu.VMEM((CHUNK, D_MODEL), jnp.float32),
            pltpu.VMEM((CHUNK, D_MODEL), jnp.float32),
            pltpu.SemaphoreType.DMA,
            pltpu.SemaphoreType.DMA,
            pltpu.SemaphoreType.DMA,
            pltpu.SemaphoreType.DMA,
            pltpu.SemaphoreType.DMA,
            pltpu.SemaphoreType.DMA,
            pltpu.SemaphoreType.DMA,
            pltpu.SemaphoreType.DMA,
            pltpu.SemaphoreType.DMA,
            pltpu.SemaphoreType.DMA,
        ],
    )
    return k(x2d, token_table, position_table)


def kernel(x, token_table, position_table):
    out = _run(x.astype(jnp.int32), token_table, position_table)
    return out.reshape(BATCH, SEQ_LEN, D_MODEL)


# speedup vs baseline: 1.3347x; 1.1002x over previous
import functools

import jax
import jax.numpy as jnp
from jax import lax
from jax.experimental import pallas as pl
from jax.experimental.pallas import tpu as pltpu
from jax.experimental.pallas import tpu_sc as plsc

VOCAB_SIZE = 100000
D_MODEL = 1024
MAX_LEN = 8192
BATCH = 4
SEQ_LEN = 4096

NUM_CORES = 2
NUM_SUBCORES = 16
NUM_WORKERS = NUM_CORES * NUM_SUBCORES
N_ROWS = BATCH * SEQ_LEN
S_BLOCK = SEQ_LEN // NUM_WORKERS
CHUNK = 8
N_PCHUNKS = S_BLOCK // CHUNK
LANES = 16
GROUPS = D_MODEL // LANES
LAST_I = N_PCHUNKS - 2


def _body(x_hbm, tok_hbm, pos_hbm, out_hbm, idxs, toks, poss, sgs, sps, sos):
    wid = lax.axis_index("s") * NUM_CORES + lax.axis_index("c")
    s_base = wid * S_BLOCK

    for b in range(BATCH):
        pltpu.sync_copy(x_hbm.at[b, pl.ds(s_base, S_BLOCK)], idxs[b])

    def start_gather(c, b, s):
        pltpu.async_copy(tok_hbm.at[idxs[b].at[pl.ds(c * CHUNK, CHUNK)]],
                         toks[4 * s + b], sgs[4 * s + b])

    def wait_gather(b, s):
        pltpu.make_async_copy(tok_hbm.at[pl.ds(0, CHUNK)],
                              toks[4 * s + b], sgs[4 * s + b]).wait()

    def start_pos(c, pb):
        pltpu.async_copy(pos_hbm.at[pl.ds(s_base + c * CHUNK, CHUNK)],
                         poss[pb], sps[pb])

    def wait_pos(pb):
        pltpu.make_async_copy(pos_hbm.at[pl.ds(0, CHUNK)], poss[pb], sps[pb]).wait()

    def start_out(c, b, s):
        pltpu.async_copy(toks[4 * s + b],
                         out_hbm.at[pl.ds(b * SEQ_LEN + s_base + c * CHUNK, CHUNK)],
                         sos[4 * s + b])

    def wait_out(b, s):
        pltpu.make_async_copy(toks[4 * s + b],
                              out_hbm.at[pl.ds(0, CHUNK)], sos[4 * s + b]).wait()

    def add_chunk(s, pb):
        bufs = [toks[4 * s + b] for b in range(BATCH)]
        pos_v = poss[pb]

        def row_add(r, _):
            for g in range(GROUPS):
                sl = pl.ds(g * LANES, LANES)
                p = pos_v[r, sl]
                for b in range(BATCH):
                    bufs[b][r, sl] = bufs[b][r, sl] + p
            return 0

        lax.fori_loop(0, CHUNK, row_add, 0, unroll=False)

    start_pos(0, 0)
    for b in range(BATCH):
        start_gather(0, b, 0)

    @pl.loop(0, N_PCHUNKS, step=2)
    def _chunk_pair(i):
        for cc in (0, 1):
            c = i + cc
            pb = cc
            s = cc
            o = 1 - cc
            if cc == 0:
                start_pos(c + 1, 1 - pb)
            else:
                @pl.when(i < LAST_I)
                def _():
                    start_pos(c + 1, 1 - pb)

            for b in range(BATCH):
                if cc == 0:
                    @pl.when(i > 0)
                    def _():
                        wait_out(b, o)

                    start_gather(c + 1, b, o)
                else:
                    wait_out(b, o)

                    @pl.when(i < LAST_I)
                    def _():
                        start_gather(c + 1, b, o)

            wait_pos(pb)
            for b in range(BATCH):
                wait_gather(b, s)
            add_chunk(s, pb)
            for b in range(BATCH):
                start_out(c, b, s)

    for b in range(BATCH):
        wait_out(b, 1)


@jax.jit
def _run(x2d, token_table, position_table):
    mesh = plsc.VectorSubcoreMesh(core_axis_name="c", subcore_axis_name="s")
    k = pl.kernel(
        _body,
        out_type=jax.ShapeDtypeStruct((N_ROWS, D_MODEL), jnp.float32),
        mesh=mesh,
        scratch_types=[
            [pltpu.VMEM((S_BLOCK,), jnp.int32) for _ in range(BATCH)],
            [pltpu.VMEM((CHUNK, D_MODEL), jnp.float32) for _ in range(8)],
            [pltpu.VMEM((CHUNK, D_MODEL), jnp.float32) for _ in range(2)],
            [pltpu.SemaphoreType.DMA for _ in range(8)],
            [pltpu.SemaphoreType.DMA for _ in range(2)],
            [pltpu.SemaphoreType.DMA for _ in range(8)],
        ],
    )
    return k(x2d, token_table, position_table)


def kernel(x, token_table, position_table):
    out = _run(x.astype(jnp.int32), token_table, position_table)
    return out.reshape(BATCH, SEQ_LEN, D_MODEL)


# speedup vs baseline: 1.3608x; 1.0195x over previous
import functools

import jax
import jax.numpy as jnp
from jax import lax
from jax.experimental import pallas as pl
from jax.experimental.pallas import tpu as pltpu
from jax.experimental.pallas import tpu_sc as plsc

VOCAB_SIZE = 100000
D_MODEL = 1024
MAX_LEN = 8192
BATCH = 4
SEQ_LEN = 4096

NUM_CORES = 2
NUM_SUBCORES = 16
NUM_WORKERS = NUM_CORES * NUM_SUBCORES
N_ROWS = BATCH * SEQ_LEN
S_BLOCK = SEQ_LEN // NUM_WORKERS
CHUNK = 8
GROW = BATCH * CHUNK
N_PCHUNKS = S_BLOCK // CHUNK
ROWS_PER_WORKER = BATCH * S_BLOCK
LANES = 16
GROUPS = D_MODEL // LANES
LAST_I = N_PCHUNKS - 2


def _body(x_hbm, tok_hbm, pos_hbm, out_hbm, idx_v, toks, poss, sgs, sps, sos):
    wid = lax.axis_index("s") * NUM_CORES + lax.axis_index("c")
    s_base = wid * S_BLOCK

    pltpu.sync_copy(x_hbm.at[pl.ds(wid * ROWS_PER_WORKER, ROWS_PER_WORKER)], idx_v)

    def start_gather(c, s):
        pltpu.async_copy(tok_hbm.at[idx_v.at[pl.ds(c * GROW, GROW)]],
                         toks[s], sgs[s])

    def wait_gather(s):
        pltpu.make_async_copy(tok_hbm.at[pl.ds(0, GROW)], toks[s], sgs[s]).wait()

    def start_pos(c, pb):
        pltpu.async_copy(pos_hbm.at[pl.ds(s_base + c * CHUNK, CHUNK)],
                         poss[pb], sps[pb])

    def wait_pos(pb):
        pltpu.make_async_copy(pos_hbm.at[pl.ds(0, CHUNK)], poss[pb], sps[pb]).wait()

    def start_outs(c, s):
        for b in range(BATCH):
            pltpu.async_copy(
                toks[s].at[pl.ds(b * CHUNK, CHUNK)],
                out_hbm.at[pl.ds(b * SEQ_LEN + s_base + c * CHUNK, CHUNK)],
                sos[s])

    def drain_outs(s):
        for _ in range(BATCH):
            pltpu.make_async_copy(toks[s].at[pl.ds(0, CHUNK)],
                                  out_hbm.at[pl.ds(0, CHUNK)], sos[s]).wait()

    def add_chunk(s, pb):
        tok_v, pos_v = toks[s], poss[pb]

        def row_add(r, _):
            for g in range(GROUPS):
                sl = pl.ds(g * LANES, LANES)
                p = pos_v[r, sl]
                for b in range(BATCH):
                    tok_v[b * CHUNK + r, sl] = tok_v[b * CHUNK + r, sl] + p
            return 0

        lax.fori_loop(0, CHUNK, row_add, 0, unroll=False)

    start_pos(0, 0)
    start_gather(0, 0)

    @pl.loop(0, N_PCHUNKS, step=2)
    def _chunk_pair(i):
        for cc in (0, 1):
            c = i + cc
            pb = cc
            s = cc
            o = 1 - cc
            if cc == 0:
                start_pos(c + 1, 1 - pb)
            else:
                @pl.when(i < LAST_I)
                def _():
                    start_pos(c + 1, 1 - pb)

            if cc == 0:
                @pl.when(i > 0)
                def _():
                    drain_outs(o)

                start_gather(c + 1, o)
            else:
                drain_outs(o)

                @pl.when(i < LAST_I)
                def _():
                    start_gather(c + 1, o)

            wait_pos(pb)
            wait_gather(s)
            add_chunk(s, pb)
            start_outs(c, s)

    drain_outs(1)


@jax.jit
def _run(x_r, token_table, position_table):
    mesh = plsc.VectorSubcoreMesh(core_axis_name="c", subcore_axis_name="s")
    k = pl.kernel(
        _body,
        out_type=jax.ShapeDtypeStruct((N_ROWS, D_MODEL), jnp.float32),
        mesh=mesh,
        scratch_types=[
            pltpu.VMEM((NUM_WORKERS * ROWS_PER_WORKER // NUM_WORKERS,), jnp.int32),
            [pltpu.VMEM((GROW, D_MODEL), jnp.float32) for _ in range(2)],
            [pltpu.VMEM((CHUNK, D_MODEL), jnp.float32) for _ in range(2)],
            [pltpu.SemaphoreType.DMA for _ in range(2)],
            [pltpu.SemaphoreType.DMA for _ in range(2)],
            [pltpu.SemaphoreType.DMA for _ in range(2)],
        ],
    )
    return k(x_r, token_table, position_table)


def kernel(x, token_table, position_table):
    x_r = (x.astype(jnp.int32)
           .reshape(BATCH, NUM_WORKERS, N_PCHUNKS, CHUNK)
           .transpose(1, 2, 0, 3)
           .reshape(N_ROWS))
    out = _run(x_r, token_table, position_table)
    return out.reshape(BATCH, SEQ_LEN, D_MODEL)
